# baseline (speedup 1.0000x reference)
import functools

import jax
import jax.numpy as jnp
from jax import lax
from jax.experimental import pallas as pl
from jax.experimental.pallas import tpu as pltpu
from jax.experimental.pallas import tpu_sc as plsc

N = 10000
E = 320000
D = 128
G = 64
NCLS = 10

NC = 2
NS = 16
NW = NC * NS

CHUNK = 128
HALF = 40
PCHUNKS = 2 * HALF
E_PAD = PCHUNKS * CHUNK * NW
PAD = E_PAD - E

R_SUB = 624
R_LAST = N - R_SUB * (NS - 1)


def _slice_phase(s, copy_chunk):
    base = s * R_SUB

    @pl.when(s < NS - 1)
    def _():
        def blk(k, _):
            copy_chunk(base + k * 104, 104)
            return _
        lax.fori_loop(0, 6, blk, 0)

    @pl.when(s == NS - 1)
    def _():
        def blk(k, _):
            copy_chunk(base + k * 128, 128)
            return _
        lax.fori_loop(0, 5, blk, 0)


@functools.cache
def _make_agg():
    mesh = plsc.VectorSubcoreMesh(core_axis_name="c", subcore_axis_name="s")
    scratch = [
        pltpu.VMEM((HALF, CHUNK), jnp.int32),
        pltpu.VMEM((HALF, CHUNK), jnp.int32),
        pltpu.VMEM((CHUNK, D), jnp.float32),
        pltpu.VMEM((CHUNK, D), jnp.float32),
        pltpu.VMEM_SHARED((N + CHUNK, D), jnp.float32),
        pltpu.SemaphoreType.DMA,
        pltpu.SemaphoreType.DMA,
    ]

    def body(h_hbm, src_hbm, dst_hbm, z128_hbm, acc_out, srcb, dstb, b0, b1,
             acc_sh, sem0, sem1):
        c = lax.axis_index("c")
        s = lax.axis_index("s")
        w = c * NS + s

        _slice_phase(s, lambda r0, nr: pltpu.sync_copy(
            z128_hbm.at[pl.ds(0, nr)], acc_sh.at[pl.ds(r0, nr)]))
        plsc.subcore_barrier()

        def gather(chunk, buf, sem):
            return pltpu.make_async_copy(h_hbm.at[srcb.at[chunk]], buf, sem)

        def scatter(chunk, buf):
            pltpu.sync_copy(buf, acc_sh.at[dstb.at[chunk]], add=True)

        for hh in range(2):
            pltpu.sync_copy(src_hbm.at[w].at[pl.ds(hh * HALF, HALF)], srcb)
            pltpu.sync_copy(dst_hbm.at[w].at[pl.ds(hh * HALF, HALF)], dstb)
            gather(0, b0, sem0).start()

            def loop_body(k, _):
                c0 = 2 * k
                gather(c0 + 1, b1, sem1).start()
                gather(c0, b0, sem0).wait()
                scatter(c0, b0)
                gather(c0 + 2, b0, sem0).start()
                gather(c0 + 1, b1, sem1).wait()
                scatter(c0 + 1, b1)
                return _

            lax.fori_loop(0, HALF // 2 - 1, loop_body, 0)
            gather(HALF - 1, b1, sem1).start()
            gather(HALF - 2, b0, sem0).wait()
            scatter(HALF - 2, b0)
            gather(HALF - 1, b1, sem1).wait()
            scatter(HALF - 1, b1)

        plsc.subcore_barrier()

        _slice_phase(s, lambda r0, nr: pltpu.sync_copy(
            acc_sh.at[pl.ds(r0, nr)], acc_out.at[c].at[pl.ds(r0, nr)]))

    return pl.kernel(body,
                     out_type=jax.ShapeDtypeStruct((NC, N, D), jnp.float32),
                     mesh=mesh, scratch_types=scratch, name="sage_agg")


def _slice_phase16(s, copy_chunk):
    base = s * R_SUB

    @pl.when(s < NS - 1)
    def _():
        def blk(k, _):
            copy_chunk(base + k * 112, 112)
            return _
        lax.fori_loop(0, 5, blk, 0)
        copy_chunk(base + 560, 64)

    @pl.when(s == NS - 1)
    def _():
        def blk(k, _):
            copy_chunk(base + k * 128, 128)
            return _
        lax.fori_loop(0, 5, blk, 0)


@functools.cache
def _make_deg():
    mesh = plsc.VectorSubcoreMesh(core_axis_name="c", subcore_axis_name="s")
    scratch = [
        pltpu.VMEM((PCHUNKS, CHUNK), jnp.int32),
        pltpu.VMEM((CHUNK, D), jnp.float32),
        pltpu.VMEM_SHARED((N + CHUNK, D), jnp.float32),
    ]

    def body(dst_hbm, z128_hbm, ones128_hbm, deg_out, dstb, onesr, deg_sh):
        c = lax.axis_index("c")
        s = lax.axis_index("s")
        w = c * NS + s

        _slice_phase(s, lambda r0, nr: pltpu.sync_copy(
            z128_hbm.at[pl.ds(0, nr)], deg_sh.at[pl.ds(r0, nr)]))
        pltpu.sync_copy(ones128_hbm, onesr)
        pltpu.sync_copy(dst_hbm.at[w], dstb)
        plsc.subcore_barrier()

        def loop_body(i, _):
            pltpu.sync_copy(onesr, deg_sh.at[dstb.at[i]], add=True)
            return _

        lax.fori_loop(0, PCHUNKS, loop_body, 0)

        plsc.subcore_barrier()

        _slice_phase(s, lambda r0, nr: pltpu.sync_copy(
            deg_sh.at[pl.ds(r0, nr)], deg_out.at[c].at[pl.ds(r0, nr)]))

    return pl.kernel(body,
                     out_type=jax.ShapeDtypeStruct((NC, N, D), jnp.float32),
                     mesh=mesh, scratch_types=scratch, name="sage_deg")


BLK = 1000
NBLK = N // BLK


def _layer0_body(a0, a1, d0, d1, h, wl, wr, b, out, rout):
    deg = (d0[:, 0:1] + d1[:, 0:1]).astype(jnp.float32)
    r = 1.0 / jnp.maximum(deg, 1.0)
    mean = (a0[...] + a1[...]) * r
    y = (jnp.dot(mean, wl[...], preferred_element_type=jnp.float32)
         + jnp.dot(h[...], wr[...], preferred_element_type=jnp.float32)
         + b[...])
    out[...] = jnp.maximum(y, 0.0)
    rout[...] = r


def _layer_body(a0, a1, r, h, wl, wr, b, out):
    mean = (a0[...] + a1[...]) * r[...]
    y = (jnp.dot(mean, wl[...], preferred_element_type=jnp.float32)
         + jnp.dot(h[...], wr[...], preferred_element_type=jnp.float32)
         + b[...])
    out[...] = jnp.maximum(y, 0.0)


def _final_body(a0, a1, r, h, wl, wr, b, batch, wout, bout, logits,
                s_scr, c_scr):
    i = pl.program_id(0)
    mean = (a0[...] + a1[...]) * r[...]
    y = (jnp.dot(mean, wl[...], preferred_element_type=jnp.float32)
         + jnp.dot(h[...], wr[...], preferred_element_type=jnp.float32)
         + b[...])
    out = jnp.maximum(y, 0.0)
    onehot = (batch[...] == lax.broadcasted_iota(jnp.int32, (1, G), 1)
              ).astype(jnp.float32)
    contract = (((0,), (0,)), ((), ()))
    s_blk = lax.dot_general(onehot, out, contract,
                            preferred_element_type=jnp.float32)
    c_blk = lax.dot_general(onehot, jnp.ones((BLK, D), jnp.float32), contract,
                            preferred_element_type=jnp.float32)

    @pl.when(i == 0)
    def _():
        s_scr[...] = jnp.zeros((G, D), jnp.float32)
        c_scr[...] = jnp.zeros((G, D), jnp.float32)

    s_scr[...] += s_blk
    c_scr[...] += c_blk

    @pl.when(i == NBLK - 1)
    def _():
        pooled = s_scr[...] / jnp.maximum(c_scr[...], 1.0)
        logits[...] = (jnp.dot(pooled, wout[...],
                               preferred_element_type=jnp.float32) + bout[...])


def _row_spec(width):
    return pl.BlockSpec((BLK, width), lambda i: (i, 0))


def _const_spec(shape):
    return pl.BlockSpec(shape, lambda i: (0, 0))


_layer0_call = pl.pallas_call(
    _layer0_body,
    grid=(NBLK,),
    in_specs=[_row_spec(D), _row_spec(D), _row_spec(D), _row_spec(D),
              _row_spec(D), _const_spec((D, D)), _const_spec((D, D)),
              _const_spec((1, D))],
    out_specs=[_row_spec(D), _row_spec(1)],
    out_shape=[jax.ShapeDtypeStruct((N, D), jnp.float32),
               jax.ShapeDtypeStruct((N, 1), jnp.float32)],
)

_layer_call = pl.pallas_call(
    _layer_body,
    grid=(NBLK,),
    in_specs=[_row_spec(D), _row_spec(D), _row_spec(1), _row_spec(D),
              _const_spec((D, D)), _const_spec((D, D)), _const_spec((1, D))],
    out_specs=_row_spec(D),
    out_shape=jax.ShapeDtypeStruct((N, D), jnp.float32),
)

_final_call = pl.pallas_call(
    _final_body,
    grid=(NBLK,),
    in_specs=[_row_spec(D), _row_spec(D), _row_spec(1), _row_spec(D),
              _const_spec((D, D)), _const_spec((D, D)), _const_spec((1, D)),
              _row_spec(1), _const_spec((D, NCLS)), _const_spec((1, NCLS))],
    out_specs=_const_spec((G, NCLS)),
    out_shape=jax.ShapeDtypeStruct((G, NCLS), jnp.float32),
    scratch_shapes=[pltpu.VMEM((G, D), jnp.float32),
                    pltpu.VMEM((G, D), jnp.float32)],
)


def kernel(x, edge_index, batch, Wl0, Wr0, b0, Wl1, Wr1, b1, Wl2, Wr2, b2,
           Wout, bout):
    spread = jnp.arange(PAD, dtype=jnp.int32) % CHUNK
    src = jnp.reshape(
        jnp.concatenate([edge_index[0], spread]), (NW, PCHUNKS, CHUNK))
    dst = jnp.reshape(
        jnp.concatenate([edge_index[1], N + spread]), (NW, PCHUNKS, CHUNK))
    z128 = jnp.zeros((CHUNK, D), jnp.float32)
    ones128 = jnp.ones((CHUNK, D), jnp.float32)
    b0r = jnp.reshape(b0, (1, D))
    b1r = jnp.reshape(b1, (1, D))
    b2r = jnp.reshape(b2, (1, D))
    boutr = jnp.reshape(bout, (1, NCLS))
    batch2 = jnp.reshape(batch, (N, 1))

    deg = _make_deg()(dst, z128, ones128)
    agg = _make_agg()

    acc = agg(x, src, dst, z128)
    h1, r = _layer0_call(acc[0], acc[1], deg[0], deg[1], x, Wl0, Wr0, b0r)
    acc1 = agg(h1, src, dst, z128)
    h2 = _layer_call(acc1[0], acc1[1], r, h1, Wl1, Wr1, b1r)
    acc2 = agg(h2, src, dst, z128)
    logits = _final_call(acc2[0], acc2[1], r, h2, Wl2, Wr2, b2r,
                         batch2, Wout, boutr)
    return logits


# speedup vs baseline: 1.0555x; 1.0555x over previous
import functools

import jax
import jax.numpy as jnp
from jax import lax
from jax.experimental import pallas as pl
from jax.experimental.pallas import tpu as pltpu
from jax.experimental.pallas import tpu_sc as plsc

N = 10000
E = 320000
D = 128
G = 64
NCLS = 10

NC = 2
NS = 16
NW = NC * NS

CHUNK = 128
HALF = 40
PCHUNKS = 2 * HALF
E_PAD = PCHUNKS * CHUNK * NW
PAD = E_PAD - E

R_SUB = 624
R_LAST = N - R_SUB * (NS - 1)


def _slice_phase(s, copy_chunk):
    base = s * R_SUB

    @pl.when(s < NS - 1)
    def _():
        def blk(k, _):
            copy_chunk(base + k * 104, 104)
            return _
        lax.fori_loop(0, 6, blk, 0)

    @pl.when(s == NS - 1)
    def _():
        def blk(k, _):
            copy_chunk(base + k * 128, 128)
            return _
        lax.fori_loop(0, 5, blk, 0)


@functools.cache
def _make_agg():
    mesh = plsc.VectorSubcoreMesh(core_axis_name="c", subcore_axis_name="s")
    scratch = [
        pltpu.VMEM((HALF, CHUNK), jnp.int32),
        pltpu.VMEM((HALF, CHUNK), jnp.int32),
        pltpu.VMEM((CHUNK, D), jnp.float32),
        pltpu.VMEM((CHUNK, D), jnp.float32),
        pltpu.VMEM_SHARED((N + CHUNK, D), jnp.float32),
        pltpu.SemaphoreType.DMA,
        pltpu.SemaphoreType.DMA,
    ]

    def body(h_hbm, src_hbm, dst_hbm, z128_hbm, acc_out, srcb, dstb, b0, b1,
             acc_sh, sem0, sem1):
        c = lax.axis_index("c")
        s = lax.axis_index("s")
        w = c * NS + s

        pltpu.sync_copy(z128_hbm, b0)
        _slice_phase(s, lambda r0, nr: pltpu.sync_copy(
            b0.at[pl.ds(0, nr)], acc_sh.at[pl.ds(r0, nr)]))
        plsc.subcore_barrier()

        def gather(chunk, buf, sem):
            return pltpu.make_async_copy(h_hbm.at[srcb.at[chunk]], buf, sem)

        def scatter(chunk, buf):
            pltpu.sync_copy(buf, acc_sh.at[dstb.at[chunk]], add=True)

        for hh in range(2):
            pltpu.sync_copy(src_hbm.at[w].at[pl.ds(hh * HALF, HALF)], srcb)
            pltpu.sync_copy(dst_hbm.at[w].at[pl.ds(hh * HALF, HALF)], dstb)
            gather(0, b0, sem0).start()

            def loop_body(k, _):
                c0 = 2 * k
                gather(c0 + 1, b1, sem1).start()
                gather(c0, b0, sem0).wait()
                scatter(c0, b0)
                gather(c0 + 2, b0, sem0).start()
                gather(c0 + 1, b1, sem1).wait()
                scatter(c0 + 1, b1)
                return _

            lax.fori_loop(0, HALF // 2 - 1, loop_body, 0)
            gather(HALF - 1, b1, sem1).start()
            gather(HALF - 2, b0, sem0).wait()
            scatter(HALF - 2, b0)
            gather(HALF - 1, b1, sem1).wait()
            scatter(HALF - 1, b1)

        plsc.subcore_barrier()

        def out_chunk(r0, nr):
            pltpu.sync_copy(acc_sh.at[pl.ds(r0, nr)], b0.at[pl.ds(0, nr)])
            pltpu.sync_copy(b0.at[pl.ds(0, nr)],
                            acc_out.at[c].at[pl.ds(r0, nr)])

        _slice_phase(s, out_chunk)

    return pl.kernel(body,
                     out_type=jax.ShapeDtypeStruct((NC, N, D), jnp.float32),
                     mesh=mesh, scratch_types=scratch, name="sage_agg")


def _slice_phase16(s, copy_chunk):
    base = s * R_SUB

    @pl.when(s < NS - 1)
    def _():
        def blk(k, _):
            copy_chunk(base + k * 112, 112)
            return _
        lax.fori_loop(0, 5, blk, 0)
        copy_chunk(base + 560, 64)

    @pl.when(s == NS - 1)
    def _():
        def blk(k, _):
            copy_chunk(base + k * 128, 128)
            return _
        lax.fori_loop(0, 5, blk, 0)


@functools.cache
def _make_deg():
    mesh = plsc.VectorSubcoreMesh(core_axis_name="c", subcore_axis_name="s")
    scratch = [
        pltpu.VMEM((PCHUNKS, CHUNK), jnp.int32),
        pltpu.VMEM((CHUNK, D), jnp.float32),
        pltpu.VMEM((CHUNK, D), jnp.float32),
        pltpu.VMEM_SHARED((N + CHUNK, D), jnp.float32),
        pltpu.SemaphoreType.DMA,
        pltpu.SemaphoreType.DMA,
    ]

    def body(dst_hbm, z128_hbm, ones128_hbm, deg_out, dstb, onesr,
             zrows, deg_sh, sem0, sem1):
        c = lax.axis_index("c")
        s = lax.axis_index("s")
        w = c * NS + s

        pltpu.sync_copy(z128_hbm, zrows)
        _slice_phase(s, lambda r0, nr: pltpu.sync_copy(
            zrows.at[pl.ds(0, nr)], deg_sh.at[pl.ds(r0, nr)]))
        pltpu.sync_copy(ones128_hbm, onesr)
        pltpu.sync_copy(dst_hbm.at[w], dstb)
        plsc.subcore_barrier()

        def dscat(i, sem):
            pltpu.async_copy(onesr, deg_sh.at[dstb.at[i]], sem, add=True)

        def dwait(i, sem):
            pltpu.make_async_copy(onesr, deg_sh.at[dstb.at[i]], sem).wait()

        dscat(0, sem0)

        def loop_body(k, _):
            c0 = 2 * k
            dscat(c0 + 1, sem1)
            dwait(c0, sem0)
            dscat(c0 + 2, sem0)
            dwait(c0 + 1, sem1)
            return _

        lax.fori_loop(0, PCHUNKS // 2 - 1, loop_body, 0)
        dscat(PCHUNKS - 1, sem1)
        dwait(PCHUNKS - 2, sem0)
        dwait(PCHUNKS - 1, sem1)

        plsc.subcore_barrier()

        def outd_chunk(r0, nr):
            pltpu.sync_copy(deg_sh.at[pl.ds(r0, nr)], zrows.at[pl.ds(0, nr)])
            pltpu.sync_copy(zrows.at[pl.ds(0, nr)],
                            deg_out.at[c].at[pl.ds(r0, nr)])

        _slice_phase(s, outd_chunk)

    return pl.kernel(body,
                     out_type=jax.ShapeDtypeStruct((NC, N, D), jnp.float32),
                     mesh=mesh, scratch_types=scratch, name="sage_deg")


BLK = 1000
NBLK = N // BLK


def _layer0_body(a0, a1, d0, d1, h, wl, wr, b, out, rout):
    deg = (d0[:, 0:1] + d1[:, 0:1]).astype(jnp.float32)
    r = 1.0 / jnp.maximum(deg, 1.0)
    mean = (a0[...] + a1[...]) * r
    y = (jnp.dot(mean, wl[...], preferred_element_type=jnp.float32)
         + jnp.dot(h[...], wr[...], preferred_element_type=jnp.float32)
         + b[...])
    out[...] = jnp.maximum(y, 0.0)
    rout[...] = r


def _layer_body(a0, a1, r, h, wl, wr, b, out):
    mean = (a0[...] + a1[...]) * r[...]
    y = (jnp.dot(mean, wl[...], preferred_element_type=jnp.float32)
         + jnp.dot(h[...], wr[...], preferred_element_type=jnp.float32)
         + b[...])
    out[...] = jnp.maximum(y, 0.0)


def _final_body(a0, a1, r, h, wl, wr, b, batch, wout, bout, logits,
                s_scr, c_scr):
    i = pl.program_id(0)
    mean = (a0[...] + a1[...]) * r[...]
    y = (jnp.dot(mean, wl[...], preferred_element_type=jnp.float32)
         + jnp.dot(h[...], wr[...], preferred_element_type=jnp.float32)
         + b[...])
    out = jnp.maximum(y, 0.0)
    onehot = (batch[...] == lax.broadcasted_iota(jnp.int32, (1, G), 1)
              ).astype(jnp.float32)
    contract = (((0,), (0,)), ((), ()))
    s_blk = lax.dot_general(onehot, out, contract,
                            preferred_element_type=jnp.float32)
    c_blk = lax.dot_general(onehot, jnp.ones((BLK, D), jnp.float32), contract,
                            preferred_element_type=jnp.float32)

    @pl.when(i == 0)
    def _():
        s_scr[...] = jnp.zeros((G, D), jnp.float32)
        c_scr[...] = jnp.zeros((G, D), jnp.float32)

    s_scr[...] += s_blk
    c_scr[...] += c_blk

    @pl.when(i == NBLK - 1)
    def _():
        pooled = s_scr[...] / jnp.maximum(c_scr[...], 1.0)
        logits[...] = (jnp.dot(pooled, wout[...],
                               preferred_element_type=jnp.float32) + bout[...])


def _row_spec(width):
    return pl.BlockSpec((BLK, width), lambda i: (i, 0))


def _const_spec(shape):
    return pl.BlockSpec(shape, lambda i: (0, 0))


_layer0_call = pl.pallas_call(
    _layer0_body,
    grid=(NBLK,),
    in_specs=[_row_spec(D), _row_spec(D), _row_spec(D), _row_spec(D),
              _row_spec(D), _const_spec((D, D)), _const_spec((D, D)),
              _const_spec((1, D))],
    out_specs=[_row_spec(D), _row_spec(1)],
    out_shape=[jax.ShapeDtypeStruct((N, D), jnp.float32),
               jax.ShapeDtypeStruct((N, 1), jnp.float32)],
)

_layer_call = pl.pallas_call(
    _layer_body,
    grid=(NBLK,),
    in_specs=[_row_spec(D), _row_spec(D), _row_spec(1), _row_spec(D),
              _const_spec((D, D)), _const_spec((D, D)), _const_spec((1, D))],
    out_specs=_row_spec(D),
    out_shape=jax.ShapeDtypeStruct((N, D), jnp.float32),
)

_final_call = pl.pallas_call(
    _final_body,
    grid=(NBLK,),
    in_specs=[_row_spec(D), _row_spec(D), _row_spec(1), _row_spec(D),
              _const_spec((D, D)), _const_spec((D, D)), _const_spec((1, D)),
              _row_spec(1), _const_spec((D, NCLS)), _const_spec((1, NCLS))],
    out_specs=_const_spec((G, NCLS)),
    out_shape=jax.ShapeDtypeStruct((G, NCLS), jnp.float32),
    scratch_shapes=[pltpu.VMEM((G, D), jnp.float32),
                    pltpu.VMEM((G, D), jnp.float32)],
)


def kernel(x, edge_index, batch, Wl0, Wr0, b0, Wl1, Wr1, b1, Wl2, Wr2, b2,
           Wout, bout):
    spread = jnp.arange(PAD, dtype=jnp.int32) % CHUNK
    src = jnp.reshape(
        jnp.concatenate([edge_index[0], spread]), (NW, PCHUNKS, CHUNK))
    dst = jnp.reshape(
        jnp.concatenate([edge_index[1], N + spread]), (NW, PCHUNKS, CHUNK))
    z128 = jnp.zeros((CHUNK, D), jnp.float32)
    ones128 = jnp.ones((CHUNK, D), jnp.float32)
    b0r = jnp.reshape(b0, (1, D))
    b1r = jnp.reshape(b1, (1, D))
    b2r = jnp.reshape(b2, (1, D))
    boutr = jnp.reshape(bout, (1, NCLS))
    batch2 = jnp.reshape(batch, (N, 1))

    deg = _make_deg()(dst, z128, ones128)
    agg = _make_agg()

    acc = agg(x, src, dst, z128)
    h1, r = _layer0_call(acc[0], acc[1], deg[0], deg[1], x, Wl0, Wr0, b0r)
    acc1 = agg(h1, src, dst, z128)
    h2 = _layer_call(acc1[0], acc1[1], r, h1, Wl1, Wr1, b1r)
    acc2 = agg(h2, src, dst, z128)
    logits = _final_call(acc2[0], acc2[1], r, h2, Wl2, Wr2, b2r,
                         batch2, Wout, boutr)
    return logits


# speedup vs baseline: 1.0692x; 1.0130x over previous
import functools

import jax
import jax.numpy as jnp
from jax import lax
from jax.experimental import pallas as pl
from jax.experimental.pallas import tpu as pltpu
from jax.experimental.pallas import tpu_sc as plsc

N = 10000
E = 320000
D = 128
G = 64
NCLS = 10

NC = 2
NS = 16
NW = NC * NS

CHUNK = 128
HALF = 40
PCHUNKS = 2 * HALF
E_PAD = PCHUNKS * CHUNK * NW
PAD = E_PAD - E

R_SUB = 624
R_LAST = N - R_SUB * (NS - 1)


def _slice_phase(s, copy_chunk):
    base = s * R_SUB

    @pl.when(s < NS - 1)
    def _():
        def blk(k, _):
            copy_chunk(base + k * 104, 104)
            return _
        lax.fori_loop(0, 6, blk, 0)

    @pl.when(s == NS - 1)
    def _():
        def blk(k, _):
            copy_chunk(base + k * 128, 128)
            return _
        lax.fori_loop(0, 5, blk, 0)


@functools.cache
def _make_agg():
    mesh = plsc.VectorSubcoreMesh(core_axis_name="c", subcore_axis_name="s")
    scratch = [
        pltpu.VMEM((HALF, CHUNK), jnp.int32),
        pltpu.VMEM((HALF, CHUNK), jnp.int32),
        pltpu.VMEM((CHUNK, D), jnp.float32),
        pltpu.VMEM((CHUNK, D), jnp.float32),
        pltpu.VMEM_SHARED((N + CHUNK, D), jnp.float32),
        pltpu.SemaphoreType.DMA,
        pltpu.SemaphoreType.DMA,
    ]

    def body(h_hbm, src_hbm, dst_hbm, z128_hbm, acc_out, srcb, dstb, b0, b1,
             acc_sh, sem0, sem1):
        c = lax.axis_index("c")
        s = lax.axis_index("s")
        w = c * NS + s

        pltpu.sync_copy(z128_hbm, b0)
        _slice_phase(s, lambda r0, nr: pltpu.sync_copy(
            b0.at[pl.ds(0, nr)], acc_sh.at[pl.ds(r0, nr)]))
        plsc.subcore_barrier()

        def gather(chunk, buf, sem):
            return pltpu.make_async_copy(h_hbm.at[srcb.at[chunk]], buf, sem)

        def scatter(chunk, buf):
            pltpu.sync_copy(buf, acc_sh.at[dstb.at[chunk]], add=True)

        for hh in range(2):
            pltpu.sync_copy(src_hbm.at[w].at[pl.ds(hh * HALF, HALF)], srcb)
            pltpu.sync_copy(dst_hbm.at[w].at[pl.ds(hh * HALF, HALF)], dstb)
            gather(0, b0, sem0).start()

            def loop_body(k, _):
                c0 = 2 * k
                gather(c0 + 1, b1, sem1).start()
                gather(c0, b0, sem0).wait()
                scatter(c0, b0)
                gather(c0 + 2, b0, sem0).start()
                gather(c0 + 1, b1, sem1).wait()
                scatter(c0 + 1, b1)
                return _

            lax.fori_loop(0, HALF // 2 - 1, loop_body, 0)
            gather(HALF - 1, b1, sem1).start()
            gather(HALF - 2, b0, sem0).wait()
            scatter(HALF - 2, b0)
            gather(HALF - 1, b1, sem1).wait()
            scatter(HALF - 1, b1)

        plsc.subcore_barrier()

        def out_chunk(r0, nr):
            pltpu.sync_copy(acc_sh.at[pl.ds(r0, nr)], b0.at[pl.ds(0, nr)])
            pltpu.sync_copy(b0.at[pl.ds(0, nr)],
                            acc_out.at[c].at[pl.ds(r0, nr)])

        _slice_phase(s, out_chunk)

    return pl.kernel(body,
                     out_type=jax.ShapeDtypeStruct((NC, N, D), jnp.float32),
                     mesh=mesh, scratch_types=scratch, name="sage_agg")


def _slice_phase16(s, copy_chunk):
    base = s * R_SUB

    @pl.when(s < NS - 1)
    def _():
        def blk(k, _):
            copy_chunk(base + k * 112, 112)
            return _
        lax.fori_loop(0, 5, blk, 0)
        copy_chunk(base + 560, 64)

    @pl.when(s == NS - 1)
    def _():
        def blk(k, _):
            copy_chunk(base + k * 128, 128)
            return _
        lax.fori_loop(0, 5, blk, 0)


@functools.cache
def _make_deg():
    mesh = plsc.VectorSubcoreMesh(core_axis_name="c", subcore_axis_name="s")
    scratch = [
        pltpu.VMEM((PCHUNKS, CHUNK), jnp.int32),
        pltpu.VMEM((CHUNK, D), jnp.float32),
        pltpu.VMEM((CHUNK, D), jnp.float32),
        pltpu.VMEM_SHARED((N + CHUNK, D), jnp.float32),
        pltpu.SemaphoreType.DMA,
        pltpu.SemaphoreType.DMA,
    ]

    def body(dst_hbm, z128_hbm, ones128_hbm, deg_out, dstb, onesr,
             zrows, deg_sh, sem0, sem1):
        c = lax.axis_index("c")
        s = lax.axis_index("s")
        w = c * NS + s

        pltpu.sync_copy(z128_hbm, zrows)
        _slice_phase(s, lambda r0, nr: pltpu.sync_copy(
            zrows.at[pl.ds(0, nr)], deg_sh.at[pl.ds(r0, nr)]))
        pltpu.sync_copy(ones128_hbm, onesr)
        pltpu.sync_copy(dst_hbm.at[w], dstb)
        plsc.subcore_barrier()

        def dscat(i, sem):
            pltpu.async_copy(onesr, deg_sh.at[dstb.at[i]], sem, add=True)

        def dwait(i, sem):
            pltpu.make_async_copy(onesr, deg_sh.at[dstb.at[i]], sem).wait()

        dscat(0, sem0)

        def loop_body(k, _):
            c0 = 2 * k
            dscat(c0 + 1, sem1)
            dwait(c0, sem0)
            dscat(c0 + 2, sem0)
            dwait(c0 + 1, sem1)
            return _

        lax.fori_loop(0, PCHUNKS // 2 - 1, loop_body, 0)
        dscat(PCHUNKS - 1, sem1)
        dwait(PCHUNKS - 2, sem0)
        dwait(PCHUNKS - 1, sem1)

        plsc.subcore_barrier()

        def outd_chunk(r0, nr):
            pltpu.sync_copy(deg_sh.at[pl.ds(r0, nr)], zrows.at[pl.ds(0, nr)])
            pltpu.sync_copy(zrows.at[pl.ds(0, nr)],
                            deg_out.at[c].at[pl.ds(r0, nr)])

        _slice_phase(s, outd_chunk)

    return pl.kernel(body,
                     out_type=jax.ShapeDtypeStruct((NC, N, D), jnp.float32),
                     mesh=mesh, scratch_types=scratch, name="sage_deg")


BLK = 2000
NBLK = N // BLK


def _layer0_body(a0, a1, d0, d1, h, wl, wr, b, out, rout):
    deg = (d0[:, 0:1] + d1[:, 0:1]).astype(jnp.float32)
    r = 1.0 / jnp.maximum(deg, 1.0)
    mean = (a0[...] + a1[...]) * r
    y = (jnp.dot(mean, wl[...], preferred_element_type=jnp.float32)
         + jnp.dot(h[...], wr[...], preferred_element_type=jnp.float32)
         + b[...])
    out[...] = jnp.maximum(y, 0.0)
    rout[...] = r


def _layer_body(a0, a1, r, h, wl, wr, b, out):
    mean = (a0[...] + a1[...]) * r[...]
    y = (jnp.dot(mean, wl[...], preferred_element_type=jnp.float32)
         + jnp.dot(h[...], wr[...], preferred_element_type=jnp.float32)
         + b[...])
    out[...] = jnp.maximum(y, 0.0)


def _final_body(a0, a1, r, h, wl, wr, b, batch, wout, bout, logits,
                s_scr, c_scr):
    i = pl.program_id(0)
    mean = (a0[...] + a1[...]) * r[...]
    y = (jnp.dot(mean, wl[...], preferred_element_type=jnp.float32)
         + jnp.dot(h[...], wr[...], preferred_element_type=jnp.float32)
         + b[...])
    out = jnp.maximum(y, 0.0)
    onehot = (batch[...] == lax.broadcasted_iota(jnp.int32, (1, G), 1)
              ).astype(jnp.float32)
    contract = (((0,), (0,)), ((), ()))
    s_blk = lax.dot_general(onehot, out, contract,
                            preferred_element_type=jnp.float32)
    c_blk = lax.dot_general(onehot, jnp.ones((BLK, D), jnp.float32), contract,
                            preferred_element_type=jnp.float32)

    @pl.when(i == 0)
    def _():
        s_scr[...] = jnp.zeros((G, D), jnp.float32)
        c_scr[...] = jnp.zeros((G, D), jnp.float32)

    s_scr[...] += s_blk
    c_scr[...] += c_blk

    @pl.when(i == NBLK - 1)
    def _():
        pooled = s_scr[...] / jnp.maximum(c_scr[...], 1.0)
        logits[...] = (jnp.dot(pooled, wout[...],
                               preferred_element_type=jnp.float32) + bout[...])


def _row_spec(width):
    return pl.BlockSpec((BLK, width), lambda i: (i, 0))


def _const_spec(shape):
    return pl.BlockSpec(shape, lambda i: (0, 0))


_layer0_call = pl.pallas_call(
    _layer0_body,
    grid=(NBLK,),
    in_specs=[_row_spec(D), _row_spec(D), _row_spec(D), _row_spec(D),
              _row_spec(D), _const_spec((D, D)), _const_spec((D, D)),
              _const_spec((1, D))],
    out_specs=[_row_spec(D), _row_spec(1)],
    out_shape=[jax.ShapeDtypeStruct((N, D), jnp.float32),
               jax.ShapeDtypeStruct((N, 1), jnp.float32)],
)

_layer_call = pl.pallas_call(
    _layer_body,
    grid=(NBLK,),
    in_specs=[_row_spec(D), _row_spec(D), _row_spec(1), _row_spec(D),
              _const_spec((D, D)), _const_spec((D, D)), _const_spec((1, D))],
    out_specs=_row_spec(D),
    out_shape=jax.ShapeDtypeStruct((N, D), jnp.float32),
)

_final_call = pl.pallas_call(
    _final_body,
    grid=(NBLK,),
    in_specs=[_row_spec(D), _row_spec(D), _row_spec(1), _row_spec(D),
              _const_spec((D, D)), _const_spec((D, D)), _const_spec((1, D)),
              _row_spec(1), _const_spec((D, NCLS)), _const_spec((1, NCLS))],
    out_specs=_const_spec((G, NCLS)),
    out_shape=jax.ShapeDtypeStruct((G, NCLS), jnp.float32),
    scratch_shapes=[pltpu.VMEM((G, D), jnp.float32),
                    pltpu.VMEM((G, D), jnp.float32)],
)


def kernel(x, edge_index, batch, Wl0, Wr0, b0, Wl1, Wr1, b1, Wl2, Wr2, b2,
           Wout, bout):
    spread = jnp.arange(PAD, dtype=jnp.int32) % CHUNK
    src = jnp.reshape(
        jnp.concatenate([edge_index[0], spread]), (NW, PCHUNKS, CHUNK))
    dst = jnp.reshape(
        jnp.concatenate([edge_index[1], N + spread]), (NW, PCHUNKS, CHUNK))
    z128 = jnp.zeros((CHUNK, D), jnp.float32)
    ones128 = jnp.ones((CHUNK, D), jnp.float32)
    b0r = jnp.reshape(b0, (1, D))
    b1r = jnp.reshape(b1, (1, D))
    b2r = jnp.reshape(b2, (1, D))
    boutr = jnp.reshape(bout, (1, NCLS))
    batch2 = jnp.reshape(batch, (N, 1))

    deg = _make_deg()(dst, z128, ones128)
    agg = _make_agg()

    acc = agg(x, src, dst, z128)
    h1, r = _layer0_call(acc[0], acc[1], deg[0], deg[1], x, Wl0, Wr0, b0r)
    acc1 = agg(h1, src, dst, z128)
    h2 = _layer_call(acc1[0], acc1[1], r, h1, Wl1, Wr1, b1r)
    acc2 = agg(h2, src, dst, z128)
    logits = _final_call(acc2[0], acc2[1], r, h2, Wl2, Wr2, b2r,
                         batch2, Wout, boutr)
    return logits
